# baseline (speedup 1.0000x reference)
import functools

import jax
import jax.numpy as jnp
from jax import lax
from jax.experimental import pallas as pl
from jax.experimental.pallas import tpu as pltpu
from jax.experimental.pallas import tpu_sc as plsc

T = 2048
D = 1024
I = 4096
E = 8
BLK = 256
CAP = ((2 * T + E * (BLK - 1) + BLK - 1) // BLK) * BLK
NB = CAP // BLK
IC = 1024
NC = I // IC
SCHUNK = 1024
NEG = -1e30


def _fiota(shape, dim):
    return jax.lax.broadcasted_iota(jnp.int32, shape, dim).astype(jnp.float32)


def _route_body(x_ref, rw_ref, tok_ref, wt_ref, be_ref):
    f32 = jnp.float32
    x = x_ref[...]
    rw = rw_ref[...]
    logits = jax.lax.dot_general(
        x, rw, (((1,), (1,)), ((), ())), preferred_element_type=f32)

    e_iota = _fiota((T, E), 1)
    m1 = jnp.max(logits, axis=1, keepdims=True)
    a1 = jnp.min(jnp.where(logits == m1, e_iota, f32(E)), axis=1,
                 keepdims=True)
    oh1 = (e_iota == a1).astype(f32)
    masked = jnp.where(oh1 > 0, f32(NEG), logits)
    m2 = jnp.max(masked, axis=1, keepdims=True)
    a2 = jnp.min(jnp.where(masked == m2, e_iota, f32(E)), axis=1,
                 keepdims=True)
    oh2 = (e_iota == a2).astype(f32)

    r = jnp.exp(m2 - m1)
    w1 = 1.0 / (1.0 + r)
    w2 = r / (1.0 + r)

    bf16 = jnp.bfloat16
    row_i = _fiota((T, 1), 0)
    col_i = _fiota((1, T), 1)
    ltri = (col_i < row_i).astype(bf16)
    ohb = jnp.concatenate([oh1.astype(bf16), oh2.astype(bf16)], axis=1)
    c12 = jax.lax.dot_general(ltri, ohb, (((1,), (0,)), ((), ())),
                              preferred_element_type=f32)
    c1 = c12[:, 0:E]
    c2 = c12[:, E:2 * E]
    cnt1 = jnp.sum(oh1, axis=0, keepdims=True)
    cnt2 = jnp.sum(oh2, axis=0, keepdims=True)
    cnt = cnt1 + cnt2

    rank1 = jnp.sum(oh1 * c1, axis=1, keepdims=True)
    rank2 = jnp.sum(oh2 * (c2 + cnt1), axis=1, keepdims=True)

    pcnt = jnp.floor((cnt + f32(BLK - 1)) * f32(1.0 / BLK)) * f32(BLK)
    ei = _fiota((E, E), 0)
    ej = _fiota((E, E), 1)
    sut = (ei < ej).astype(f32)
    pstart = jax.lax.dot_general(pcnt, sut, (((1,), (0,)), ((), ())),
                                 preferred_element_type=f32)
    pend = pstart + pcnt

    pos1 = jnp.sum(oh1 * pstart, axis=1, keepdims=True) + rank1
    pos2 = jnp.sum(oh2 * pstart, axis=1, keepdims=True) + rank2

    bstart = _fiota((1, NB), 1) * f32(BLK)
    be = jnp.zeros((1, NB), f32)
    for e in range(E):
        ps = pstart[0:1, e:e + 1]
        pe_ = pend[0:1, e:e + 1]
        be = be + f32(e) * ((bstart >= ps) & (bstart < pe_)).astype(f32)
    be_ref[0:1, 0:NB] = be.astype(jnp.int32)
    nbu = jnp.sum(pcnt, axis=1, keepdims=True) * f32(1.0 / BLK)
    be_ref[0:1, NB:NB + 1] = nbu.astype(jnp.int32)

    t_col = _fiota((T, 1), 0)
    posc = jnp.concatenate([pos1, pos2], axis=0)
    tokc = jnp.concatenate([t_col, t_col], axis=0)
    thi = jnp.floor(tokc * f32(1.0 / 32.0))
    tlo = tokc - thi * f32(32.0)
    wtc = jnp.concatenate([w1, w2], axis=0)
    whi = wtc.astype(bf16)
    wlo = (wtc - whi.astype(f32)).astype(bf16)
    rhs4 = jnp.concatenate(
        [thi.astype(bf16), tlo.astype(bf16), whi, wlo], axis=1)
    slot_i = _fiota((1, SCHUNK), 1)
    for c in range(CAP // SCHUNK):
        m = (posc == (slot_i + f32(c * SCHUNK))).astype(bf16)
        r = jax.lax.dot_general(m, rhs4, (((0,), (0,)), ((), ())),
                                preferred_element_type=f32)
        tok_ref[c * SCHUNK:(c + 1) * SCHUNK, :] = (
            r[:, 0:1] * f32(32.0) + r[:, 1:2])
        wt_ref[c * SCHUNK:(c + 1) * SCHUNK, :] = r[:, 2:3] + r[:, 3:4]


_SC_NW = 32
_SC_GB = CAP // _SC_NW
_SC_CH = 2
_SC_GC = _SC_GB // _SC_CH


def _sc_gather_body(x_hbm, idx_hbm, out_hbm, idx_v, rows_v, sem):
    wid = lax.axis_index("s") * 2 + lax.axis_index("c")
    base = wid * _SC_GB
    for ch in range(_SC_CH):
        off = base + ch * _SC_GC
        pltpu.sync_copy(idx_hbm.at[pl.ds(off, _SC_GC)], idx_v)
        pltpu.async_copy(x_hbm.at[idx_v], rows_v, sem).wait()
        pltpu.sync_copy(rows_v, out_hbm.at[pl.ds(off, _SC_GC)])


def _sc_gather(x, idx):
    mesh = plsc.VectorSubcoreMesh(core_axis_name="c", subcore_axis_name="s")
    return pl.kernel(
        _sc_gather_body,
        mesh=mesh,
        out_type=jax.ShapeDtypeStruct((CAP, D), jnp.float32),
        scratch_types=[
            pltpu.VMEM((_SC_GC,), jnp.int32),
            pltpu.VMEM((_SC_GC, D), jnp.float32),
            pltpu.SemaphoreType.DMA,
        ],
    )(x, idx)


def _moe_body(be_ref, xs_ref, wsg_ref, wsu_ref, w2_ref, tok_ref, wt_ref,
              out_ref, ya_ref):
    f32 = jnp.float32
    b = pl.program_id(0)
    c = pl.program_id(1)

    @pl.when(b < be_ref[NB])
    def _used_block():
        xb = xs_ref[...]
        gate = jax.lax.dot_general(xb, wsg_ref[0], (((1,), (1,)), ((), ())),
                                   preferred_element_type=f32)
        up = jax.lax.dot_general(xb, wsu_ref[0], (((1,), (1,)), ((), ())),
                                 preferred_element_type=f32)
        h = gate * up / (1.0 + jnp.exp(-gate))
        yc = jax.lax.dot_general(h, w2_ref[0], (((1,), (1,)), ((), ())),
                                 preferred_element_type=f32)

        @pl.when(c == 0)
        def _init():
            ya_ref[...] = yc

        @pl.when(c > 0)
        def _acc():
            ya_ref[...] = ya_ref[...] + yc

        @pl.when(c == NC - 1)
        def _combine():
            ids = tok_ref[0]
            t_i = _fiota((T, BLK), 0)
            s = (t_i == ids).astype(f32) * wt_ref[0]
            contrib = jax.lax.dot_general(
                s, ya_ref[...], (((1,), (0,)), ((), ())),
                preferred_element_type=f32)

            @pl.when(b == 0)
            def _first_out():
                out_ref[...] = contrib

            @pl.when(b > 0)
            def _acc_out():
                out_ref[...] = out_ref[...] + contrib


@functools.partial(jax.jit)
def kernel(hidden_states, router_w, ws, w2s):
    f32 = jnp.float32
    x = hidden_states.astype(f32)

    tok, wt, be = pl.pallas_call(
        _route_body,
        out_shape=[
            jax.ShapeDtypeStruct((CAP, 1), f32),
            jax.ShapeDtypeStruct((CAP, 1), f32),
            jax.ShapeDtypeStruct((1, NB + 1), jnp.int32),
        ],
    )(x, router_w.astype(f32))

    tok3 = jnp.reshape(tok, (NB, 1, BLK))
    wt3 = jnp.reshape(wt, (NB, 1, BLK))
    be1 = jnp.reshape(be, (NB + 1,))

    xs = _sc_gather(x, jnp.reshape(tok, (CAP,)).astype(jnp.int32))

    def _wsg_map(b, c, be):
        u = b < be[NB]
        e = jnp.where(u, be[b], be[be[NB] - 1])
        return (e, jnp.where(u, c, NC - 1), 0)

    def _wsu_map(b, c, be):
        u = b < be[NB]
        e = jnp.where(u, be[b], be[be[NB] - 1])
        return (e, jnp.where(u, NC + c, 2 * NC - 1), 0)

    def _w2_map(b, c, be):
        u = b < be[NB]
        e = jnp.where(u, be[b], be[be[NB] - 1])
        return (e, 0, jnp.where(u, c, NC - 1))

    grid_spec = pltpu.PrefetchScalarGridSpec(
        num_scalar_prefetch=1,
        grid=(NB, NC),
        in_specs=[
            pl.BlockSpec((BLK, D), lambda b, c, be: (b, 0)),
            pl.BlockSpec((1, IC, D), _wsg_map),
            pl.BlockSpec((1, IC, D), _wsu_map),
            pl.BlockSpec((1, D, IC), _w2_map),
            pl.BlockSpec((1, 1, BLK), lambda b, c, be: (b, 0, 0)),
            pl.BlockSpec((1, 1, BLK), lambda b, c, be: (b, 0, 0)),
        ],
        out_specs=pl.BlockSpec((T, D), lambda b, c, be: (0, 0)),
        scratch_shapes=[
            pltpu.VMEM((BLK, D), f32),
        ],
    )
    out = pl.pallas_call(
        _moe_body,
        grid_spec=grid_spec,
        out_shape=jax.ShapeDtypeStruct((T, D), f32),
    )(be1, xs, ws.astype(f32), ws.astype(f32), w2s.astype(f32), tok3, wt3)
    return out


# speedup vs baseline: 1.1839x; 1.1839x over previous
import functools

import jax
import jax.numpy as jnp
from jax.experimental import pallas as pl
from jax.experimental.pallas import tpu as pltpu

T = 2048
D = 1024
I = 4096
E = 8
BLK = 256
CAP = ((2 * T + E * (BLK - 1) + BLK - 1) // BLK) * BLK
NB = CAP // BLK
IC = 512
NC = I // IC
SCHUNK = 1024
NEG = -1e30


def _fiota(shape, dim):
    return jax.lax.broadcasted_iota(jnp.int32, shape, dim).astype(jnp.float32)


def _route_body(x_ref, rw_ref, tok_ref, wt_ref, be_ref):
    f32 = jnp.float32
    x = x_ref[...]
    rw = rw_ref[...]
    logits = jax.lax.dot_general(
        x, rw, (((1,), (1,)), ((), ())), preferred_element_type=f32)

    e_iota = _fiota((T, E), 1)
    m1 = jnp.max(logits, axis=1, keepdims=True)
    a1 = jnp.min(jnp.where(logits == m1, e_iota, f32(E)), axis=1,
                 keepdims=True)
    oh1 = (e_iota == a1).astype(f32)
    masked = jnp.where(oh1 > 0, f32(NEG), logits)
    m2 = jnp.max(masked, axis=1, keepdims=True)
    a2 = jnp.min(jnp.where(masked == m2, e_iota, f32(E)), axis=1,
                 keepdims=True)
    oh2 = (e_iota == a2).astype(f32)

    r = jnp.exp(m2 - m1)
    w1 = 1.0 / (1.0 + r)
    w2 = r / (1.0 + r)

    bf16 = jnp.bfloat16
    row_i = _fiota((T, 1), 0)
    col_i = _fiota((1, T), 1)
    ltri = (col_i < row_i).astype(bf16)
    ohb = jnp.concatenate([oh1.astype(bf16), oh2.astype(bf16)], axis=1)
    c12 = jax.lax.dot_general(ltri, ohb, (((1,), (0,)), ((), ())),
                              preferred_element_type=f32)
    c1 = c12[:, 0:E]
    c2 = c12[:, E:2 * E]
    cnt1 = jnp.sum(oh1, axis=0, keepdims=True)
    cnt2 = jnp.sum(oh2, axis=0, keepdims=True)
    cnt = cnt1 + cnt2

    rank1 = jnp.sum(oh1 * c1, axis=1, keepdims=True)
    rank2 = jnp.sum(oh2 * (c2 + cnt1), axis=1, keepdims=True)

    pcnt = jnp.floor((cnt + f32(BLK - 1)) * f32(1.0 / BLK)) * f32(BLK)
    ei = _fiota((E, E), 0)
    ej = _fiota((E, E), 1)
    sut = (ei < ej).astype(f32)
    pstart = jax.lax.dot_general(pcnt, sut, (((1,), (0,)), ((), ())),
                                 preferred_element_type=f32)
    pend = pstart + pcnt

    pos1 = jnp.sum(oh1 * pstart, axis=1, keepdims=True) + rank1
    pos2 = jnp.sum(oh2 * pstart, axis=1, keepdims=True) + rank2

    bstart = _fiota((1, NB), 1) * f32(BLK)
    be = jnp.zeros((1, NB), f32)
    for e in range(E):
        ps = pstart[0:1, e:e + 1]
        pe_ = pend[0:1, e:e + 1]
        be = be + f32(e) * ((bstart >= ps) & (bstart < pe_)).astype(f32)
    be_ref[0:1, 0:NB] = be.astype(jnp.int32)
    nbu = jnp.sum(pcnt, axis=1, keepdims=True) * f32(1.0 / BLK)
    be_ref[0:1, NB:NB + 1] = nbu.astype(jnp.int32)

    t_col = _fiota((T, 1), 0)
    posc = jnp.concatenate([pos1, pos2], axis=0)
    tokc = jnp.concatenate([t_col, t_col], axis=0)
    thi = jnp.floor(tokc * f32(1.0 / 32.0))
    tlo = tokc - thi * f32(32.0)
    wtc = jnp.concatenate([w1, w2], axis=0)
    whi = wtc.astype(bf16)
    wlo = (wtc - whi.astype(f32)).astype(bf16)
    rhs4 = jnp.concatenate(
        [thi.astype(bf16), tlo.astype(bf16), whi, wlo], axis=1)
    slot_i = _fiota((1, SCHUNK), 1)
    for c in range(CAP // SCHUNK):
        m = (posc == (slot_i + f32(c * SCHUNK))).astype(bf16)
        r = jax.lax.dot_general(m, rhs4, (((0,), (0,)), ((), ())),
                                preferred_element_type=f32)
        tok_ref[c * SCHUNK:(c + 1) * SCHUNK, :] = (
            r[:, 0:1] * f32(32.0) + r[:, 1:2])
        wt_ref[c * SCHUNK:(c + 1) * SCHUNK, :] = r[:, 2:3] + r[:, 3:4]


def _moe_body(be_ref, x_ref, wsg_ref, wsu_ref, w2_ref, tok_ref, wt_ref,
              out_ref, xb_ref, ya_ref, gt_ref):
    f32 = jnp.float32
    b = pl.program_id(0)
    c = pl.program_id(1)

    @pl.when(b < be_ref[NB])
    def _used_block():
        @pl.when(c == 0)
        def _gather():
            ids = tok_ref[0]
            t_i = _fiota((T, BLK), 0)
            gt = (t_i == ids).astype(f32)
            gt_ref[...] = gt
            xb_ref[...] = jax.lax.dot_general(
                gt, x_ref[...], (((0,), (0,)), ((), ())),
                preferred_element_type=f32)

        xb = xb_ref[...]
        gate = jax.lax.dot_general(xb, wsg_ref[0], (((1,), (1,)), ((), ())),
                                   preferred_element_type=f32)
        up = jax.lax.dot_general(xb, wsu_ref[0], (((1,), (1,)), ((), ())),
                                 preferred_element_type=f32)
        h = gate * up / (1.0 + jnp.exp(-gate))
        yc = jax.lax.dot_general(h, w2_ref[0], (((1,), (1,)), ((), ())),
                                 preferred_element_type=f32)

        @pl.when(c == 0)
        def _init():
            ya_ref[...] = yc

        @pl.when(c > 0)
        def _acc():
            ya_ref[...] = ya_ref[...] + yc

        @pl.when(c == NC - 1)
        def _combine():
            s = gt_ref[...] * wt_ref[0]
            contrib = jax.lax.dot_general(
                s, ya_ref[...], (((1,), (0,)), ((), ())),
                preferred_element_type=f32)

            @pl.when(b == 0)
            def _first_out():
                out_ref[...] = contrib

            @pl.when(b > 0)
            def _acc_out():
                out_ref[...] = out_ref[...] + contrib


@functools.partial(jax.jit)
def kernel(hidden_states, router_w, ws, w2s):
    f32 = jnp.float32
    x = hidden_states.astype(f32)

    tok, wt, be = pl.pallas_call(
        _route_body,
        out_shape=[
            jax.ShapeDtypeStruct((CAP, 1), f32),
            jax.ShapeDtypeStruct((CAP, 1), f32),
            jax.ShapeDtypeStruct((1, NB + 1), jnp.int32),
        ],
    )(x, router_w.astype(f32))

    tok3 = jnp.reshape(tok, (NB, 1, BLK))
    wt3 = jnp.reshape(wt, (NB, 1, BLK))
    be1 = jnp.reshape(be, (NB + 1,))

    def _wsg_map(b, c, be):
        u = b < be[NB]
        e = jnp.where(u, be[b], be[be[NB] - 1])
        return (e, jnp.where(u, c, NC - 1), 0)

    def _wsu_map(b, c, be):
        u = b < be[NB]
        e = jnp.where(u, be[b], be[be[NB] - 1])
        return (e, jnp.where(u, NC + c, 2 * NC - 1), 0)

    def _w2_map(b, c, be):
        u = b < be[NB]
        e = jnp.where(u, be[b], be[be[NB] - 1])
        return (e, 0, jnp.where(u, c, NC - 1))

    grid_spec = pltpu.PrefetchScalarGridSpec(
        num_scalar_prefetch=1,
        grid=(NB, NC),
        in_specs=[
            pl.BlockSpec((T, D), lambda b, c, be: (0, 0)),
            pl.BlockSpec((1, IC, D), _wsg_map),
            pl.BlockSpec((1, IC, D), _wsu_map),
            pl.BlockSpec((1, D, IC), _w2_map),
            pl.BlockSpec((1, 1, BLK), lambda b, c, be: (b, 0, 0)),
            pl.BlockSpec((1, 1, BLK), lambda b, c, be: (b, 0, 0)),
        ],
        out_specs=pl.BlockSpec((T, D), lambda b, c, be: (0, 0)),
        scratch_shapes=[
            pltpu.VMEM((BLK, D), f32),
            pltpu.VMEM((BLK, D), f32),
            pltpu.VMEM((T, BLK), f32),
        ],
    )
    out = pl.pallas_call(
        _moe_body,
        grid_spec=grid_spec,
        out_shape=jax.ShapeDtypeStruct((T, D), f32),
    )(be1, x, ws.astype(f32), ws.astype(f32), w2s.astype(f32), tok3, wt3)
    return out


# speedup vs baseline: 1.3500x; 1.1403x over previous
import functools

import jax
import jax.numpy as jnp
from jax.experimental import pallas as pl
from jax.experimental.pallas import tpu as pltpu

T = 2048
D = 1024
I = 4096
E = 8
BLK = 256
CAP = ((2 * T + E * (BLK - 1) + BLK - 1) // BLK) * BLK
NB = CAP // BLK
IC = 1024
NC = I // IC
SCHUNK = 1024
NEG = -1e30


def _fiota(shape, dim):
    return jax.lax.broadcasted_iota(jnp.int32, shape, dim).astype(jnp.float32)


def _route_body(x_ref, rw_ref, tok_ref, wt_ref, be_ref):
    f32 = jnp.float32
    x = x_ref[...]
    rw = rw_ref[...]
    logits = jax.lax.dot_general(
        x, rw, (((1,), (1,)), ((), ())), preferred_element_type=f32)

    e_iota = _fiota((T, E), 1)
    m1 = jnp.max(logits, axis=1, keepdims=True)
    a1 = jnp.min(jnp.where(logits == m1, e_iota, f32(E)), axis=1,
                 keepdims=True)
    oh1 = (e_iota == a1).astype(f32)
    masked = jnp.where(oh1 > 0, f32(NEG), logits)
    m2 = jnp.max(masked, axis=1, keepdims=True)
    a2 = jnp.min(jnp.where(masked == m2, e_iota, f32(E)), axis=1,
                 keepdims=True)
    oh2 = (e_iota == a2).astype(f32)

    r = jnp.exp(m2 - m1)
    w1 = 1.0 / (1.0 + r)
    w2 = r / (1.0 + r)

    bf16 = jnp.bfloat16
    row_i = _fiota((T, 1), 0)
    col_i = _fiota((1, T), 1)
    ltri = (col_i < row_i).astype(bf16)
    ohb = jnp.concatenate([oh1.astype(bf16), oh2.astype(bf16)], axis=1)
    c12 = jax.lax.dot_general(ltri, ohb, (((1,), (0,)), ((), ())),
                              preferred_element_type=f32)
    c1 = c12[:, 0:E]
    c2 = c12[:, E:2 * E]
    cnt1 = jnp.sum(oh1, axis=0, keepdims=True)
    cnt2 = jnp.sum(oh2, axis=0, keepdims=True)
    cnt = cnt1 + cnt2

    rank1 = jnp.sum(oh1 * c1, axis=1, keepdims=True)
    rank2 = jnp.sum(oh2 * (c2 + cnt1), axis=1, keepdims=True)

    pcnt = jnp.floor((cnt + f32(BLK - 1)) * f32(1.0 / BLK)) * f32(BLK)
    ei = _fiota((E, E), 0)
    ej = _fiota((E, E), 1)
    sut = (ei < ej).astype(f32)
    pstart = jax.lax.dot_general(pcnt, sut, (((1,), (0,)), ((), ())),
                                 preferred_element_type=f32)
    pend = pstart + pcnt

    pos1 = jnp.sum(oh1 * pstart, axis=1, keepdims=True) + rank1
    pos2 = jnp.sum(oh2 * pstart, axis=1, keepdims=True) + rank2

    bstart = _fiota((1, NB), 1) * f32(BLK)
    be = jnp.zeros((1, NB), f32)
    for e in range(E):
        ps = pstart[0:1, e:e + 1]
        pe_ = pend[0:1, e:e + 1]
        be = be + f32(e) * ((bstart >= ps) & (bstart < pe_)).astype(f32)
    be_ref[0:1, 0:NB] = be.astype(jnp.int32)
    nbu = jnp.sum(pcnt, axis=1, keepdims=True) * f32(1.0 / BLK)
    be_ref[0:1, NB:NB + 1] = nbu.astype(jnp.int32)

    t_col = _fiota((T, 1), 0)
    posc = jnp.concatenate([pos1, pos2], axis=0)
    tokc = jnp.concatenate([t_col, t_col], axis=0)
    thi = jnp.floor(tokc * f32(1.0 / 32.0))
    tlo = tokc - thi * f32(32.0)
    wtc = jnp.concatenate([w1, w2], axis=0)
    whi = wtc.astype(bf16)
    wlo = (wtc - whi.astype(f32)).astype(bf16)
    rhs4 = jnp.concatenate(
        [thi.astype(bf16), tlo.astype(bf16), whi, wlo], axis=1)
    slot_i = _fiota((1, SCHUNK), 1)
    for c in range(CAP // SCHUNK):
        m = (posc == (slot_i + f32(c * SCHUNK))).astype(bf16)
        r = jax.lax.dot_general(m, rhs4, (((0,), (0,)), ((), ())),
                                preferred_element_type=f32)
        tok_ref[c * SCHUNK:(c + 1) * SCHUNK, :] = (
            r[:, 0:1] * f32(32.0) + r[:, 1:2])
        wt_ref[c * SCHUNK:(c + 1) * SCHUNK, :] = r[:, 2:3] + r[:, 3:4]


def _moe_body(be_ref, x_ref, wsg_ref, wsu_ref, w2_ref, tok_ref, wt_ref,
              out_ref, xb_ref, ya_ref, gt_ref):
    f32 = jnp.float32
    b = pl.program_id(0)
    c = pl.program_id(1)

    @pl.when(b < be_ref[NB])
    def _used_block():
        @pl.when(c == 0)
        def _gather():
            ids = tok_ref[0]
            t_i = _fiota((T, BLK), 0)
            gt = (t_i == ids).astype(f32)
            gt_ref[...] = gt
            xb_ref[...] = jax.lax.dot_general(
                gt, x_ref[...], (((0,), (0,)), ((), ())),
                preferred_element_type=f32)

        xb = xb_ref[...]
        gate = jax.lax.dot_general(xb, wsg_ref[0], (((1,), (1,)), ((), ())),
                                   preferred_element_type=f32)
        up = jax.lax.dot_general(xb, wsu_ref[0], (((1,), (1,)), ((), ())),
                                 preferred_element_type=f32)
        h = gate * up / (1.0 + jnp.exp(-gate))
        yc = jax.lax.dot_general(h, w2_ref[0], (((1,), (1,)), ((), ())),
                                 preferred_element_type=f32)

        @pl.when(c == 0)
        def _init():
            ya_ref[...] = yc

        @pl.when(c > 0)
        def _acc():
            ya_ref[...] = ya_ref[...] + yc

        @pl.when(c == NC - 1)
        def _combine():
            s = gt_ref[...] * wt_ref[0]
            contrib = jax.lax.dot_general(
                s, ya_ref[...], (((1,), (0,)), ((), ())),
                preferred_element_type=f32)

            @pl.when(b == 0)
            def _first_out():
                out_ref[...] = contrib

            @pl.when(b > 0)
            def _acc_out():
                out_ref[...] = out_ref[...] + contrib


@functools.partial(jax.jit)
def kernel(hidden_states, router_w, ws, w2s):
    f32 = jnp.float32
    x = hidden_states.astype(f32)

    tok, wt, be = pl.pallas_call(
        _route_body,
        out_shape=[
            jax.ShapeDtypeStruct((CAP, 1), f32),
            jax.ShapeDtypeStruct((CAP, 1), f32),
            jax.ShapeDtypeStruct((1, NB + 1), jnp.int32),
        ],
    )(x, router_w.astype(f32))

    tok3 = jnp.reshape(tok, (NB, 1, BLK))
    wt3 = jnp.reshape(wt, (NB, 1, BLK))
    be1 = jnp.reshape(be, (NB + 1,))

    def _wsg_map(b, c, be):
        u = b < be[NB]
        e = jnp.where(u, be[b], be[be[NB] - 1])
        return (e, jnp.where(u, c, NC - 1), 0)

    def _wsu_map(b, c, be):
        u = b < be[NB]
        e = jnp.where(u, be[b], be[be[NB] - 1])
        return (e, jnp.where(u, NC + c, 2 * NC - 1), 0)

    def _w2_map(b, c, be):
        u = b < be[NB]
        e = jnp.where(u, be[b], be[be[NB] - 1])
        return (e, 0, jnp.where(u, c, NC - 1))

    grid_spec = pltpu.PrefetchScalarGridSpec(
        num_scalar_prefetch=1,
        grid=(NB, NC),
        in_specs=[
            pl.BlockSpec((T, D), lambda b, c, be: (0, 0)),
            pl.BlockSpec((1, IC, D), _wsg_map),
            pl.BlockSpec((1, IC, D), _wsu_map),
            pl.BlockSpec((1, D, IC), _w2_map),
            pl.BlockSpec((1, 1, BLK), lambda b, c, be: (b, 0, 0)),
            pl.BlockSpec((1, 1, BLK), lambda b, c, be: (b, 0, 0)),
        ],
        out_specs=pl.BlockSpec((T, D), lambda b, c, be: (0, 0)),
        scratch_shapes=[
            pltpu.VMEM((BLK, D), f32),
            pltpu.VMEM((BLK, D), f32),
            pltpu.VMEM((T, BLK), f32),
        ],
    )
    out = pl.pallas_call(
        _moe_body,
        grid_spec=grid_spec,
        out_shape=jax.ShapeDtypeStruct((T, D), f32),
    )(be1, x, ws.astype(f32), ws.astype(f32), w2s.astype(f32), tok3, wt3)
    return out
